# Initial kernel scaffold; baseline (speedup 1.0000x reference)
import jax
import jax.numpy as jnp
from jax.experimental import pallas as pl


def kernel(x, v, index, channels):
    raise NotImplementedError("write your pallas kernel here")


# speedup vs baseline: 23.8019x; 23.8019x over previous
import functools

import jax
import jax.numpy as jnp
import numpy as np
from jax.experimental import pallas as pl

N = 32768
D = 3
T = 16
RESOLUTION = 64
RADIUS = 1.0
SCALE = 8.0
MAX_CHANNELS = 4
BATCH_LEN = 16
NUM_SEG = BATCH_LEN * MAX_CHANNELS

CHUNK = 2048
NUM_BLOCKS = N // CHUNK

def _ect_kernel(x_ref, v_ref, idx_ref, out_ref):
    step = pl.program_id(0)

    x = x_ref[...]
    v = v_ref[...]
    nh = jnp.dot(x, v, preferred_element_type=jnp.float32)

    r_iota = jax.lax.broadcasted_iota(jnp.int32, (1, RESOLUTION), 1).astype(jnp.float32)
    lin = r_iota * (2.0 * RADIUS / (RESOLUTION - 1)) - RADIUS
    pieces = []
    for t in range(T):
        arg = SCALE * (lin - nh[:, t : t + 1])
        pieces.append(jax.nn.sigmoid(arg))
    ecc = jnp.concatenate(pieces, axis=1)

    idx = idx_ref[0]
    seg = jax.lax.broadcasted_iota(jnp.int32, (NUM_SEG, CHUNK), 0)
    onehot = (idx == seg).astype(jnp.float32)

    contrib = jnp.dot(onehot, ecc, preferred_element_type=jnp.float32)

    @pl.when(step == 0)
    def _init():
        out_ref[...] = contrib

    @pl.when(step > 0)
    def _acc():
        out_ref[...] = out_ref[...] + contrib

    @pl.when(step == NUM_BLOCKS - 1)
    def _normalize():
        acc = out_ref[...]
        m = jnp.max(acc, axis=1, keepdims=True)
        m = jnp.where(m == 0.0, 1.0, m)
        out_ref[...] = acc / m


@jax.jit
def kernel(x, v, index, channels):
    idx = (MAX_CHANNELS * index + channels).astype(jnp.int32)
    idx3 = idx.reshape(NUM_BLOCKS, 1, CHUNK)

    out = pl.pallas_call(
        _ect_kernel,
        grid=(NUM_BLOCKS,),
        in_specs=[
            pl.BlockSpec((CHUNK, D), lambda i: (i, 0)),
            pl.BlockSpec((D, T), lambda i: (0, 0)),
            pl.BlockSpec((1, 1, CHUNK), lambda i: (i, 0, 0)),
        ],
        out_specs=pl.BlockSpec((NUM_SEG, T * RESOLUTION), lambda i: (0, 0)),
        out_shape=jax.ShapeDtypeStruct((NUM_SEG, T * RESOLUTION), jnp.float32),
    )(x, v, idx3)

    ect = out.reshape(BATCH_LEN, MAX_CHANNELS, T, RESOLUTION)
    return jnp.transpose(ect, (0, 1, 3, 2))


# speedup vs baseline: 52.6050x; 2.2101x over previous
import math

import jax
import jax.numpy as jnp
import numpy as np
from jax.experimental import pallas as pl

N = 32768
D = 3
T = 16
RESOLUTION = 64
RADIUS = 1.0
SCALE = 8.0
MAX_CHANNELS = 4
BATCH_LEN = 16
NUM_SEG = BATCH_LEN * MAX_CHANNELS

CHUNK = 2048
NUM_BLOCKS = N // CHUNK

_A = SCALE * math.log2(math.e)
_CLAMP = 126.0

_LIN = np.linspace(-RADIUS, RADIUS, RESOLUTION).astype(np.float64)
_S = np.zeros((T, T * RESOLUTION), dtype=np.float32)
for _t in range(T):
    _S[_t, _t * RESOLUTION : (_t + 1) * RESOLUTION] = np.exp2(-_A * _LIN)


def _ect_kernel(x_ref, v2_ref, s_ref, idx_ref, out_ref):
    step = pl.program_id(0)

    x = x_ref[...]
    v2 = v2_ref[...]
    m = jnp.dot(x, v2, preferred_element_type=jnp.float32)
    m = jnp.clip(m, -_CLAMP, _CLAMP)
    e = jnp.exp2(m)

    p = jnp.dot(e, s_ref[...], preferred_element_type=jnp.float32)
    sig = 1.0 / (1.0 + p)

    idx = idx_ref[0]
    seg = jax.lax.broadcasted_iota(jnp.int32, (NUM_SEG, CHUNK), 0)
    onehot = (idx == seg).astype(jnp.float32)

    contrib = jnp.dot(onehot, sig, preferred_element_type=jnp.float32)

    @pl.when(step == 0)
    def _init():
        out_ref[...] = contrib

    @pl.when(step > 0)
    def _acc():
        out_ref[...] = out_ref[...] + contrib

    @pl.when(step == NUM_BLOCKS - 1)
    def _normalize():
        acc = out_ref[...]
        mx = jnp.max(acc, axis=1, keepdims=True)
        mx = jnp.where(mx == 0.0, 1.0, mx)
        out_ref[...] = acc / mx


@jax.jit
def kernel(x, v, index, channels):
    idx = (MAX_CHANNELS * index + channels).astype(jnp.int32)
    idx3 = idx.reshape(NUM_BLOCKS, 1, CHUNK)
    v2 = _A * v
    s = jnp.asarray(_S)

    out = pl.pallas_call(
        _ect_kernel,
        grid=(NUM_BLOCKS,),
        in_specs=[
            pl.BlockSpec((CHUNK, D), lambda i: (i, 0)),
            pl.BlockSpec((D, T), lambda i: (0, 0)),
            pl.BlockSpec((T, T * RESOLUTION), lambda i: (0, 0)),
            pl.BlockSpec((1, 1, CHUNK), lambda i: (i, 0, 0)),
        ],
        out_specs=pl.BlockSpec((NUM_SEG, T * RESOLUTION), lambda i: (0, 0)),
        out_shape=jax.ShapeDtypeStruct((NUM_SEG, T * RESOLUTION), jnp.float32),
    )(x, v2, s, idx3)

    ect = out.reshape(BATCH_LEN, MAX_CHANNELS, T, RESOLUTION)
    return jnp.transpose(ect, (0, 1, 3, 2))
